# baseline (speedup 1.0000x reference)
import functools

import jax
import jax.numpy as jnp
from jax import lax
from jax.experimental import pallas as pl
from jax.experimental.pallas import tpu as pltpu
from jax.experimental.pallas import tpu_sc as plsc

_MAX_OBS = 512
_DIM = 64
_B = 4096
_L = 200
_NC = 2
_NS = 16
_NW = _NC * _NS
_RPW = _B // _NW
_LPAD = 208


def _emb_body(tokf_hbm, emb_hbm, posx_hbm, out_hbm,
              tok0_v, pidx_v, add_v, idx_all, row0, row1, g0, g1, s0, s1):
    wid = lax.axis_index("s") * _NC + lax.axis_index("c")
    base = wid * _RPW

    tok0_v[pl.ds(192, 16)] = jnp.full((16,), -1, jnp.int32)
    pltpu.sync_copy(tokf_hbm.at[pl.ds(0, _L)], tok0_v.at[pl.ds(0, _L)])
    ii = lax.iota(jnp.int32, 16)
    best_k = jnp.full((16,), -1, jnp.int32)
    for c in range(_LPAD // 16):
        v = tok0_v[pl.ds(c * 16, 16)]
        k = (v * 256) + (207 - (ii + c * 16))
        best_k = jnp.maximum(best_k, k)
    sk, _sv = plsc.sort_key_val(best_k, best_k, descending=True)
    top = lax.squeeze(lax.slice(sk, (0,), (1,)), (0,))
    obs = 207 - (top & 255)

    for c in range(_LPAD // 16):
        p = ii + c * 16
        pidx_v[pl.ds(c * 16, 16)] = jnp.where(
            p < obs, p, jnp.where(p > obs, jnp.int32(_MAX_OBS), jnp.int32(_MAX_OBS + 1)))

    a0 = pltpu.async_copy(posx_hbm.at[pidx_v.at[pl.ds(0, 128)]],
                          add_v.at[pl.ds(0, 128)], g0)
    a1 = pltpu.async_copy(posx_hbm.at[pidx_v.at[pl.ds(128, 72)]],
                          add_v.at[pl.ds(128, 72)], g0)
    a0.wait()
    a1.wait()

    pltpu.sync_copy(tokf_hbm.at[pl.ds(base * _L, _RPW * _L)], idx_all)

    def start_gather(r, row, g):
        pltpu.async_copy(emb_hbm.at[idx_all.at[pl.ds(r * _L, 128)]],
                         row.at[pl.ds(0, 128)], g)
        pltpu.async_copy(emb_hbm.at[idx_all.at[pl.ds(r * _L + 128, 72)]],
                         row.at[pl.ds(128, 72)], g)

    def wait_gather(row, g):
        pltpu.make_async_copy(emb_hbm.at[pl.ds(0, _L)], row, g).wait()

    def start_store(r, row, s):
        pltpu.async_copy(row, out_hbm.at[pl.ds((base + r) * _L, _L)], s)

    def wait_store(row, s):
        pltpu.make_async_copy(row, out_hbm.at[pl.ds(0, _L)], s).wait()

    def add_rows(row):
        def add_l(l, c2):
            for u in range(2):
                for d in range(_DIM // 16):
                    plsc.addupdate(row.at[2 * l + u, pl.ds(d * 16, 16)],
                                   add_v[2 * l + u, pl.ds(d * 16, 16)])
            return c2
        lax.fori_loop(0, _L // 2, add_l, 0)

    start_gather(0, row0, g0)
    wait_gather(row0, g0)
    start_gather(1, row1, g1)
    add_rows(row0)
    start_store(0, row0, s0)

    def pair_step(rr, carry):
        r1 = 2 * rr + 1
        wait_gather(row1, g1)
        wait_store(row0, s0)
        start_gather(r1 + 1, row0, g0)
        add_rows(row1)
        start_store(r1, row1, s1)

        wait_gather(row0, g0)
        wait_store(row1, s1)
        start_gather(r1 + 2, row1, g1)
        add_rows(row0)
        start_store(r1 + 1, row0, s0)
        return carry
    lax.fori_loop(0, (_RPW - 2) // 2, pair_step, 0)

    wait_gather(row1, g1)
    wait_store(row0, s0)
    add_rows(row1)
    start_store(_RPW - 1, row1, s1)
    wait_store(row1, s1)


@functools.partial(
    pl.kernel,
    out_type=jax.ShapeDtypeStruct((_B * _L, _DIM), jnp.float32),
    mesh=plsc.VectorSubcoreMesh(core_axis_name="c", subcore_axis_name="s"),
    compiler_params=pltpu.CompilerParams(
        needs_layout_passes=False, use_tc_tiling_on_sc=False),
    scratch_types=[
        pltpu.VMEM((_LPAD,), jnp.int32),
        pltpu.VMEM((_LPAD,), jnp.int32),
        pltpu.VMEM((_L, _DIM), jnp.float32),
        pltpu.VMEM((_RPW * _L,), jnp.int32),
        pltpu.VMEM((_L, _DIM), jnp.float32),
        pltpu.VMEM((_L, _DIM), jnp.float32),
        pltpu.SemaphoreType.DMA,
        pltpu.SemaphoreType.DMA,
        pltpu.SemaphoreType.DMA,
        pltpu.SemaphoreType.DMA,
    ],
)
def _emb_kernel(tokf_hbm, emb_hbm, posx_hbm, out_hbm, *rest):
    _emb_body(tokf_hbm, emb_hbm, posx_hbm, out_hbm, *rest)


def kernel(tokens, emb_table, pos_table):
    posx = jnp.concatenate(
        [pos_table, jnp.zeros((1, _DIM), jnp.float32)], axis=0)
    out = _emb_kernel(tokens.reshape(-1), emb_table, posx)
    return out.reshape(_B, _L, _DIM)


# speedup vs baseline: 1.0012x; 1.0012x over previous
import functools

import jax
import jax.numpy as jnp
from jax import lax
from jax.experimental import pallas as pl
from jax.experimental.pallas import tpu as pltpu
from jax.experimental.pallas import tpu_sc as plsc

_MAX_OBS = 512
_DIM = 64
_B = 4096
_L = 200
_NC = 2
_NS = 16
_NW = _NC * _NS
_RPW = _B // _NW
_LPAD = 208


def _emb_body(tok_hbm, emb_hbm, pos_hbm, out_hbm,
              tok0_v, pidx_v, add_v, idx_all, row0, row1, g0, g1, s0, s1):
    wid = lax.axis_index("s") * _NC + lax.axis_index("c")
    base = wid * _RPW

    tok0_v[pl.ds(192, 16)] = jnp.full((16,), -1, jnp.int32)
    pltpu.sync_copy(tok_hbm.at[0], tok0_v.at[pl.ds(0, _L)])
    ii = lax.iota(jnp.int32, 16)
    best_k = jnp.full((16,), -1, jnp.int32)
    for c in range(_LPAD // 16):
        v = tok0_v[pl.ds(c * 16, 16)]
        k = (v * 256) + (207 - (ii + c * 16))
        best_k = jnp.maximum(best_k, k)
    sk, _sv = plsc.sort_key_val(best_k, best_k, descending=True)
    top = lax.squeeze(lax.slice(sk, (0,), (1,)), (0,))
    obs = 207 - (top & 255)

    for c in range(_LPAD // 16):
        p = ii + c * 16
        pidx_v[pl.ds(c * 16, 16)] = jnp.where(p < obs, p, jnp.int32(_MAX_OBS))

    a0 = pltpu.async_copy(pos_hbm.at[pidx_v.at[pl.ds(0, 128)]],
                          add_v.at[pl.ds(0, 128)], g0)
    a1 = pltpu.async_copy(pos_hbm.at[pidx_v.at[pl.ds(128, 72)]],
                          add_v.at[pl.ds(128, 72)], g0)
    a0.wait()
    a1.wait()
    for d in range(_DIM // 16):
        add_v[obs, pl.ds(d * 16, 16)] = jnp.zeros((16,), jnp.float32)

    pltpu.sync_copy(tok_hbm.at[pl.ds(base, _RPW)], idx_all)

    def start_gather(r, row, g):
        pltpu.async_copy(emb_hbm.at[idx_all.at[r, pl.ds(0, 128)]],
                         row.at[pl.ds(0, 128)], g)
        pltpu.async_copy(emb_hbm.at[idx_all.at[r, pl.ds(128, 72)]],
                         row.at[pl.ds(128, 72)], g)

    def wait_gather(row, g):
        pltpu.make_async_copy(emb_hbm.at[pl.ds(0, _L)], row, g).wait()

    def start_store(r, row, s):
        pltpu.async_copy(row, out_hbm.at[pl.ds((base + r) * _L, _L)], s)

    def wait_store(row, s):
        pltpu.make_async_copy(row, out_hbm.at[pl.ds(0, _L)], s).wait()

    def add_rows(row):
        def add_l(l, c2):
            for u in range(2):
                for d in range(_DIM // 16):
                    plsc.addupdate(row.at[2 * l + u, pl.ds(d * 16, 16)],
                                   add_v[2 * l + u, pl.ds(d * 16, 16)])
            return c2
        lax.fori_loop(0, _L // 2, add_l, 0)

    start_gather(0, row0, g0)
    wait_gather(row0, g0)
    start_gather(1, row1, g1)
    add_rows(row0)
    start_store(0, row0, s0)

    def pair_step(rr, carry):
        r1 = 2 * rr + 1
        wait_gather(row1, g1)
        wait_store(row0, s0)
        start_gather(r1 + 1, row0, g0)
        add_rows(row1)
        start_store(r1, row1, s1)

        wait_gather(row0, g0)
        wait_store(row1, s1)
        start_gather(r1 + 2, row1, g1)
        add_rows(row0)
        start_store(r1 + 1, row0, s0)
        return carry
    lax.fori_loop(0, (_RPW - 2) // 2, pair_step, 0)

    wait_gather(row1, g1)
    wait_store(row0, s0)
    add_rows(row1)
    start_store(_RPW - 1, row1, s1)
    wait_store(row1, s1)


@functools.partial(
    pl.kernel,
    out_type=jax.ShapeDtypeStruct((_B * _L, _DIM), jnp.float32),
    mesh=plsc.VectorSubcoreMesh(core_axis_name="c", subcore_axis_name="s"),
    compiler_params=pltpu.CompilerParams(
        needs_layout_passes=False, use_tc_tiling_on_sc=False),
    scratch_types=[
        pltpu.VMEM((_LPAD,), jnp.int32),
        pltpu.VMEM((_LPAD,), jnp.int32),
        pltpu.VMEM((_L, _DIM), jnp.float32),
        pltpu.VMEM((_RPW, _L), jnp.int32),
        pltpu.VMEM((_L, _DIM), jnp.float32),
        pltpu.VMEM((_L, _DIM), jnp.float32),
        pltpu.SemaphoreType.DMA,
        pltpu.SemaphoreType.DMA,
        pltpu.SemaphoreType.DMA,
        pltpu.SemaphoreType.DMA,
    ],
)
def _emb_kernel(tok_hbm, emb_hbm, pos_hbm, out_hbm, *rest):
    _emb_body(tok_hbm, emb_hbm, pos_hbm, out_hbm, *rest)


def kernel(tokens, emb_table, pos_table):
    out = _emb_kernel(tokens, emb_table, pos_table)
    return out.reshape(_B, _L, _DIM)


# speedup vs baseline: 1.0511x; 1.0499x over previous
import functools

import jax
import jax.numpy as jnp
from jax import lax
from jax.experimental import pallas as pl
from jax.experimental.pallas import tpu as pltpu
from jax.experimental.pallas import tpu_sc as plsc

_MAX_OBS = 512
_DIM = 64
_B = 4096
_L = 200
_NC = 2
_NS = 16
_NW = _NC * _NS
_RPW = _B // _NW
_LPAD = 208


def _emb_body(tok_hbm, emb_hbm, pos_hbm, out_hbm,
              tok0_v, pidx_v, add_v, idx_all,
              row0, row1, row2, row3, g0, g1, g2, g3, s0, s1, s2, s3):
    wid = lax.axis_index("s") * _NC + lax.axis_index("c")
    base = wid * _RPW

    tok0_v[pl.ds(192, 16)] = jnp.full((16,), -1, jnp.int32)
    pltpu.sync_copy(tok_hbm.at[0], tok0_v.at[pl.ds(0, _L)])
    ii = lax.iota(jnp.int32, 16)
    best_k = jnp.full((16,), -1, jnp.int32)
    for c in range(_LPAD // 16):
        v = tok0_v[pl.ds(c * 16, 16)]
        k = (v * 256) + (207 - (ii + c * 16))
        best_k = jnp.maximum(best_k, k)
    sk, _sv = plsc.sort_key_val(best_k, best_k, descending=True)
    top = lax.squeeze(lax.slice(sk, (0,), (1,)), (0,))
    obs = 207 - (top & 255)

    for c in range(_LPAD // 16):
        p = ii + c * 16
        pidx_v[pl.ds(c * 16, 16)] = jnp.where(p < obs, p, jnp.int32(_MAX_OBS))

    a0 = pltpu.async_copy(pos_hbm.at[pidx_v.at[pl.ds(0, 128)]],
                          add_v.at[pl.ds(0, 128)], g0)
    a1 = pltpu.async_copy(pos_hbm.at[pidx_v.at[pl.ds(128, 72)]],
                          add_v.at[pl.ds(128, 72)], g0)
    a0.wait()
    a1.wait()
    for d in range(_DIM // 16):
        add_v[obs, pl.ds(d * 16, 16)] = jnp.zeros((16,), jnp.float32)

    pltpu.sync_copy(tok_hbm.at[pl.ds(base, _RPW)], idx_all)

    rows = (row0, row1, row2, row3)
    gs = (g0, g1, g2, g3)
    ss = (s0, s1, s2, s3)

    def start_gather(r, row, g):
        pltpu.async_copy(emb_hbm.at[idx_all.at[r, pl.ds(0, 128)]],
                         row.at[pl.ds(0, 128)], g)
        pltpu.async_copy(emb_hbm.at[idx_all.at[r, pl.ds(128, 72)]],
                         row.at[pl.ds(128, 72)], g)

    def wait_gather(row, g):
        pltpu.make_async_copy(emb_hbm.at[pl.ds(0, _L)], row, g).wait()

    def start_store(r, row, s):
        pltpu.async_copy(row, out_hbm.at[pl.ds((base + r) * _L, _L)], s)

    def wait_store(row, s):
        pltpu.make_async_copy(row, out_hbm.at[pl.ds(0, _L)], s).wait()

    def add_rows(row):
        def add_l(l, c2):
            for u in range(2):
                for d in range(_DIM // 16):
                    plsc.addupdate(row.at[2 * l + u, pl.ds(d * 16, 16)],
                                   add_v[2 * l + u, pl.ds(d * 16, 16)])
            return c2
        lax.fori_loop(0, _L // 2, add_l, 0)

    start_gather(0, row0, g0)
    start_gather(1, row1, g1)
    start_gather(2, row2, g2)
    wait_gather(row0, g0)
    add_rows(row0)
    start_store(0, row0, s0)
    start_gather(3, row3, g3)

    def quad_step(k, carry):
        r = 4 * k + 1
        for j in range(4):
            b = (j + 1) % 4
            nb = j
            wait_gather(rows[b], gs[b])
            add_rows(rows[b])
            start_store(r + j, rows[b], ss[b])
            wait_store(rows[nb], ss[nb])
            start_gather(r + j + 3, rows[nb], gs[nb])
        return carry
    lax.fori_loop(0, (_RPW - 4) // 4, quad_step, 0)

    for j in range(3):
        b = (j + 1) % 4
        wait_gather(rows[b], gs[b])
        add_rows(rows[b])
        start_store(_RPW - 3 + j, rows[b], ss[b])
    wait_store(rows[0], ss[0])
    for j in range(3):
        b = (j + 1) % 4
        wait_store(rows[b], ss[b])


@functools.partial(
    pl.kernel,
    out_type=jax.ShapeDtypeStruct((_B * _L, _DIM), jnp.float32),
    mesh=plsc.VectorSubcoreMesh(core_axis_name="c", subcore_axis_name="s"),
    compiler_params=pltpu.CompilerParams(
        needs_layout_passes=False, use_tc_tiling_on_sc=False),
    scratch_types=[
        pltpu.VMEM((_LPAD,), jnp.int32),
        pltpu.VMEM((_LPAD,), jnp.int32),
        pltpu.VMEM((_L, _DIM), jnp.float32),
        pltpu.VMEM((_RPW, _L), jnp.int32),
        pltpu.VMEM((_L, _DIM), jnp.float32),
        pltpu.VMEM((_L, _DIM), jnp.float32),
        pltpu.VMEM((_L, _DIM), jnp.float32),
        pltpu.VMEM((_L, _DIM), jnp.float32),
        pltpu.SemaphoreType.DMA,
        pltpu.SemaphoreType.DMA,
        pltpu.SemaphoreType.DMA,
        pltpu.SemaphoreType.DMA,
        pltpu.SemaphoreType.DMA,
        pltpu.SemaphoreType.DMA,
        pltpu.SemaphoreType.DMA,
        pltpu.SemaphoreType.DMA,
    ],
)
def _emb_kernel(tok_hbm, emb_hbm, pos_hbm, out_hbm, *rest):
    _emb_body(tok_hbm, emb_hbm, pos_hbm, out_hbm, *rest)


def kernel(tokens, emb_table, pos_table):
    out = _emb_kernel(tokens, emb_table, pos_table)
    return out.reshape(_B, _L, _DIM)
